# baseline (speedup 1.0000x reference)
import functools

import jax
import jax.numpy as jnp
from jax import lax
from jax.experimental import pallas as pl
from jax.experimental.pallas import tpu as pltpu
from jax.experimental.pallas import tpu_sc as plsc

NUM_CLASSES = 1000
CONTEXT_LENGTH = 77
EMBED_DIM = 512

_B = NUM_CLASSES * CONTEXT_LENGTH
_C = 88
_NC = 2
_NS = 16
_NW = _NC * _NS
_NCHUNKS = _B // _C
_KMIN = _NCHUNKS // _NW
_EXTRA = _NCHUNKS % _NW
_KMAX = _KMIN + (1 if _EXTRA else 0)
_NBUF = 2
_BPW = _KMAX * _C
_IDX_PAD = _C * (_KMIN * (_NW - 1) + min(_NW - 1, _EXTRA)) + _BPW


def _build_gather(vocab, dim):
    mesh = plsc.VectorSubcoreMesh(
        core_axis_name="c", subcore_axis_name="s",
        num_cores=_NC, num_subcores=_NS,
    )

    @functools.partial(
        pl.kernel,
        out_type=jax.ShapeDtypeStruct((_B, dim), jnp.float32),
        mesh=mesh,
        scratch_types=[
            pltpu.VMEM((_BPW,), jnp.int32),
            pltpu.VMEM((_NBUF, _C, dim), jnp.float32),
            [pltpu.SemaphoreType.DMA] * _NBUF,
            [pltpu.SemaphoreType.DMA] * _NBUF,
        ],
    )
    def gather_kernel(table_hbm, idx_hbm, out_hbm, idx_v, rows_v,
                      gsems, osems):
        wid = lax.axis_index("s") * _NC + lax.axis_index("c")
        base = _C * (_KMIN * wid + jnp.minimum(wid, _EXTRA))
        n_valid = _KMIN + (wid < _EXTRA).astype(jnp.int32)

        pltpu.sync_copy(idx_hbm.at[pl.ds(base, _BPW)], idx_v)

        def gather_desc(k, buf, sem):
            return pltpu.make_async_copy(
                table_hbm.at[idx_v.at[pl.ds(k * _C, _C)]], rows_v.at[buf],
                sem)

        def out_desc(k, buf, sem):
            return pltpu.make_async_copy(
                rows_v.at[buf], out_hbm.at[pl.ds(base + k * _C, _C)], sem)

        def ring(p, _):
            for b in range(_NBUF):
                k = _NBUF * p + b

                @pl.when(k < n_valid)
                def _():
                    @pl.when(k >= _NBUF)
                    def _():
                        out_desc(k - _NBUF, b, osems[b]).wait()
                    gather_desc(k, b, gsems[b]).start()

            for b in range(_NBUF):
                k = _NBUF * p + b

                @pl.when(k < n_valid)
                def _():
                    gather_desc(k, b, gsems[b]).wait()
                    out_desc(k, b, osems[b]).start()

            return 0

        lax.fori_loop(0, (_KMAX + _NBUF - 1) // _NBUF, ring, 0,
                      unroll=False)

        for b in range(_NBUF):
            out_desc(0, b, osems[b]).wait()

    return gather_kernel


_gather = _build_gather(49408, EMBED_DIM)


def kernel(global_tokenized_prompts, token_embedding):
    idx = global_tokenized_prompts.astype(jnp.int32).T.reshape(-1)
    idx = jnp.concatenate(
        [idx, jnp.zeros((_IDX_PAD - _B,), jnp.int32)])
    out = _gather(token_embedding, idx)
    return out.reshape(CONTEXT_LENGTH, NUM_CLASSES, EMBED_DIM).transpose(
        1, 0, 2)


# speedup vs baseline: 1.0227x; 1.0227x over previous
import functools

import jax
import jax.numpy as jnp
from jax import lax
from jax.experimental import pallas as pl
from jax.experimental.pallas import tpu as pltpu
from jax.experimental.pallas import tpu_sc as plsc

NUM_CLASSES = 1000
CONTEXT_LENGTH = 77
EMBED_DIM = 512

_B = NUM_CLASSES * CONTEXT_LENGTH
_C = 40
_NC = 2
_NS = 16
_NW = _NC * _NS
_NCHUNKS = _B // _C
_KMIN = _NCHUNKS // _NW
_EXTRA = _NCHUNKS % _NW
_KMAX = _KMIN + (1 if _EXTRA else 0)
_NBUF = 6
_BPW = _KMAX * _C
_IDX_PAD = _C * (_KMIN * (_NW - 1) + min(_NW - 1, _EXTRA)) + _BPW


def _build_gather(vocab, dim):
    mesh = plsc.VectorSubcoreMesh(
        core_axis_name="c", subcore_axis_name="s",
        num_cores=_NC, num_subcores=_NS,
    )

    @functools.partial(
        pl.kernel,
        out_type=jax.ShapeDtypeStruct((_B, dim), jnp.float32),
        mesh=mesh,
        scratch_types=[
            pltpu.VMEM((_BPW,), jnp.int32),
            pltpu.VMEM((_NBUF, _C, dim), jnp.float32),
            [pltpu.SemaphoreType.DMA] * _NBUF,
            [pltpu.SemaphoreType.DMA] * _NBUF,
        ],
    )
    def gather_kernel(table_hbm, idx_hbm, out_hbm, idx_v, rows_v,
                      gsems, osems):
        wid = lax.axis_index("s") * _NC + lax.axis_index("c")
        base = _C * (_KMIN * wid + jnp.minimum(wid, _EXTRA))
        n_valid = _KMIN + (wid < _EXTRA).astype(jnp.int32)

        pltpu.sync_copy(idx_hbm.at[pl.ds(base, _BPW)], idx_v)

        def gather_desc(k, buf, sem):
            return pltpu.make_async_copy(
                table_hbm.at[idx_v.at[pl.ds(k * _C, _C)]], rows_v.at[buf],
                sem)

        def out_desc(k, buf, sem):
            return pltpu.make_async_copy(
                rows_v.at[buf], out_hbm.at[pl.ds(base + k * _C, _C)], sem)

        def ring(p, _):
            for b in range(_NBUF):
                k = _NBUF * p + b

                @pl.when(k < n_valid)
                def _():
                    @pl.when(k >= _NBUF)
                    def _():
                        out_desc(k - _NBUF, b, osems[b]).wait()
                    gather_desc(k, b, gsems[b]).start()

            for b in range(_NBUF):
                k = _NBUF * p + b

                @pl.when(k < n_valid)
                def _():
                    gather_desc(k, b, gsems[b]).wait()
                    out_desc(k, b, osems[b]).start()

            return 0

        lax.fori_loop(0, (_KMAX + _NBUF - 1) // _NBUF, ring, 0,
                      unroll=False)

        for b in range(_NBUF):
            out_desc(0, b, osems[b]).wait()

    return gather_kernel


_gather = _build_gather(49408, EMBED_DIM)


def kernel(global_tokenized_prompts, token_embedding):
    idx = global_tokenized_prompts.astype(jnp.int32).T.reshape(-1)
    idx = jnp.concatenate(
        [idx, jnp.zeros((_IDX_PAD - _B,), jnp.int32)])
    out = _gather(token_embedding, idx)
    return out.reshape(CONTEXT_LENGTH, NUM_CLASSES, EMBED_DIM).transpose(
        1, 0, 2)


# speedup vs baseline: 1.0448x; 1.0216x over previous
import functools

import jax
import jax.numpy as jnp
from jax import lax
from jax.experimental import pallas as pl
from jax.experimental.pallas import tpu as pltpu
from jax.experimental.pallas import tpu_sc as plsc

NUM_CLASSES = 1000
CONTEXT_LENGTH = 77
EMBED_DIM = 512

_B = NUM_CLASSES * CONTEXT_LENGTH
_C = 40
_NC = 2
_NS = 16
_NW = _NC * _NS
_NCHUNKS = _B // _C
_KMIN = _NCHUNKS // _NW
_EXTRA = _NCHUNKS % _NW
_KMAX = _KMIN + (1 if _EXTRA else 0)
_NBUF = 6
_LOOK = 3
_BPW = _KMAX * _C
_IDX_PAD = _C * (_KMIN * (_NW - 1) + min(_NW - 1, _EXTRA)) + _BPW


def _build_gather(vocab, dim):
    mesh = plsc.VectorSubcoreMesh(
        core_axis_name="c", subcore_axis_name="s",
        num_cores=_NC, num_subcores=_NS,
    )

    @functools.partial(
        pl.kernel,
        out_type=jax.ShapeDtypeStruct((_B, dim), jnp.float32),
        mesh=mesh,
        scratch_types=[
            pltpu.VMEM((_BPW,), jnp.int32),
            pltpu.VMEM((_NBUF, _C, dim), jnp.float32),
            [pltpu.SemaphoreType.DMA] * _NBUF,
            [pltpu.SemaphoreType.DMA] * _NBUF,
        ],
    )
    def gather_kernel(table_hbm, idx_hbm, out_hbm, idx_v, rows_v,
                      gsems, osems):
        wid = lax.axis_index("s") * _NC + lax.axis_index("c")
        base = _C * (_KMIN * wid + jnp.minimum(wid, _EXTRA))
        n_valid = _KMIN + (wid < _EXTRA).astype(jnp.int32)

        pltpu.sync_copy(idx_hbm.at[pl.ds(base, _BPW)], idx_v)

        def gather_desc(k, buf, sem):
            return pltpu.make_async_copy(
                table_hbm.at[idx_v.at[pl.ds(k * _C, _C)]], rows_v.at[buf],
                sem)

        def out_desc(k, buf, sem):
            return pltpu.make_async_copy(
                rows_v.at[buf], out_hbm.at[pl.ds(base + k * _C, _C)], sem)

        for c in range(_LOOK):
            gather_desc(c, c % _NBUF, gsems[c % _NBUF]).start()

        def ring(p, _):
            for b in range(_NBUF):
                k = _NBUF * p + b
                b2 = (b + _LOOK) % _NBUF

                @pl.when(k + _LOOK < n_valid)
                def _():
                    @pl.when(k + _LOOK >= _NBUF)
                    def _():
                        out_desc(k + _LOOK - _NBUF, b2, osems[b2]).wait()
                    gather_desc(k + _LOOK, b2, gsems[b2]).start()

                @pl.when(k < n_valid)
                def _():
                    gather_desc(k, b, gsems[b]).wait()
                    out_desc(k, b, osems[b]).start()

            return 0

        lax.fori_loop(0, (_KMAX + _NBUF - 1) // _NBUF, ring, 0,
                      unroll=False)

        for b in range(_NBUF):
            out_desc(0, b, osems[b]).wait()

    return gather_kernel


_gather = _build_gather(49408, EMBED_DIM)


def kernel(global_tokenized_prompts, token_embedding):
    idx = global_tokenized_prompts.astype(jnp.int32).T.reshape(-1)
    idx = jnp.concatenate(
        [idx, jnp.zeros((_IDX_PAD - _B,), jnp.int32)])
    out = _gather(token_embedding, idx)
    return out.reshape(CONTEXT_LENGTH, NUM_CLASSES, EMBED_DIM).transpose(
        1, 0, 2)
